# baseline (speedup 1.0000x reference)
import functools

import jax
import jax.numpy as jnp
from jax import lax
from jax.experimental import pallas as pl
from jax.experimental.pallas import tpu as pltpu
from jax.experimental.pallas import tpu_sc as plsc

B, S, V = 64, 8, 100000
R = B * S
NC, NS, L = 2, 16, 16
NW = NC * NS
ROWS_PER_W = R // NW
CH = 10000
NCH = V // CH
VECS = CH // L

_NEG_INF = float("-inf")


def _sc_body(x_hbm, id_hbm, val_hbm, buf0, buf1, val_mat, idx_mat, oid, oval,
             sem0, sem1):
    wid = lax.axis_index("s") * NC + lax.axis_index("c")
    row0 = wid * ROWS_PER_W
    bufs = (buf0, buf1)
    sems = (sem0, sem1)

    def start(r, c, slot):
        pltpu.make_async_copy(x_hbm.at[row0 + r, c], bufs[slot], sems[slot]).start()

    def wait(slot):
        pltpu.make_async_copy(x_hbm.at[row0, 0], bufs[slot], sems[slot]).wait()

    start(0, 0, 0)

    def row_body(r, _):
        best = jnp.full((L,), _NEG_INF, jnp.float32)
        bidx = jnp.zeros((L,), jnp.int32)
        for c in range(NCH):
            slot = c % 2
            wait(slot)
            if c + 1 < NCH:
                start(r, c + 1, slot ^ 1)
            else:
                @pl.when(r + 1 < ROWS_PER_W)
                def _prefetch():
                    start(r + 1, 0, slot ^ 1)
            buf = bufs[slot]
            idxv = lax.broadcasted_iota(jnp.int32, (L,), 0) + c * CH

            def elem(i, carry):
                b, bi, iv = carry
                x = buf[pl.ds(i * L, L)]
                m = x > b
                b = jnp.maximum(b, x)
                bi = jnp.where(m, iv, bi)
                return b, bi, iv + L

            best, bidx, _ = lax.fori_loop(
                0, VECS, elem, (best, bidx, idxv), unroll=8)

        val_mat[r, :] = best
        idx_mat[r, :] = bidx
        return 0

    lax.fori_loop(0, ROWS_PER_W, row_body, 0)

    lane = lax.broadcasted_iota(jnp.int32, (L,), 0)
    best = plsc.load_gather(val_mat, [lane, jnp.zeros((L,), jnp.int32)])
    bidx = plsc.load_gather(idx_mat, [lane, jnp.zeros((L,), jnp.int32)])
    for j in range(1, L):
        col = jnp.full((L,), j, jnp.int32)
        bv = plsc.load_gather(val_mat, [lane, col])
        bi = plsc.load_gather(idx_mat, [lane, col])
        better = (bv > best) | ((bv == best) & (bi < bidx))
        best = jnp.where(better, bv, best)
        bidx = jnp.where(better, bi, bidx)
    oid[...] = bidx
    oval[...] = best
    pltpu.sync_copy(oid, id_hbm.at[pl.ds(row0, ROWS_PER_W)])
    pltpu.sync_copy(oval, val_hbm.at[pl.ds(row0, ROWS_PER_W)])


@jax.jit
def kernel(logits):
    x = logits.reshape(R, NCH, CH)
    mesh = plsc.VectorSubcoreMesh(
        core_axis_name="c", subcore_axis_name="s", num_cores=NC, num_subcores=NS)
    run = pl.kernel(
        _sc_body,
        out_type=(
            jax.ShapeDtypeStruct((R,), jnp.int32),
            jax.ShapeDtypeStruct((R,), jnp.float32),
        ),
        mesh=mesh,
        compiler_params=pltpu.CompilerParams(needs_layout_passes=False),
        scratch_types=(
            pltpu.VMEM((CH,), jnp.float32),
            pltpu.VMEM((CH,), jnp.float32),
            pltpu.VMEM((ROWS_PER_W, L), jnp.float32),
            pltpu.VMEM((ROWS_PER_W, L), jnp.int32),
            pltpu.VMEM((ROWS_PER_W,), jnp.int32),
            pltpu.VMEM((ROWS_PER_W,), jnp.float32),
            pltpu.SemaphoreType.DMA,
            pltpu.SemaphoreType.DMA,
        ),
    )
    token_id, token_logit = run(x)
    return token_id.reshape(B, S), token_logit.reshape(B, S)


# speedup vs baseline: 1.0008x; 1.0008x over previous
import functools

import jax
import jax.numpy as jnp
from jax import lax
from jax.experimental import pallas as pl
from jax.experimental.pallas import tpu as pltpu
from jax.experimental.pallas import tpu_sc as plsc

B, S, V = 64, 8, 100000
R = B * S
NC, NS, L = 2, 16, 16
NW = NC * NS
ROWS_PER_W = R // NW
CH = 10000
NCH = V // CH
VECS = CH // L
K = 5

_NEG_INF = float("-inf")


def _sc_body(x_hbm, id_hbm, val_hbm, buf0, buf1, val_mat, idx_mat, oid, oval,
             sem0, sem1):
    wid = lax.axis_index("s") * NC + lax.axis_index("c")
    row0 = wid * ROWS_PER_W
    bufs = (buf0, buf1)
    sems = (sem0, sem1)

    def start(r, c, slot):
        pltpu.make_async_copy(x_hbm.at[row0 + r, c], bufs[slot], sems[slot]).start()

    def wait(slot):
        pltpu.make_async_copy(x_hbm.at[row0, 0], bufs[slot], sems[slot]).wait()

    start(0, 0, 0)

    lane = lax.broadcasted_iota(jnp.int32, (L,), 0)

    def row_body(r, _):
        bests = tuple(jnp.full((L,), _NEG_INF, jnp.float32) for _ in range(K))
        bidxs = tuple(jnp.zeros((L,), jnp.int32) for _ in range(K))
        civ = jnp.zeros((L,), jnp.int32)
        for c in range(NCH):
            slot = c % 2
            wait(slot)
            if c + 1 < NCH:
                start(r, c + 1, slot ^ 1)
            else:
                @pl.when(r + 1 < ROWS_PER_W)
                def _prefetch():
                    start(r + 1, 0, slot ^ 1)
            buf = bufs[slot]

            def elem(i, carry):
                bs, bis, cv = carry
                bs, bis = list(bs), list(bis)
                for k in range(K):
                    x = buf[pl.ds((i * K + k) * L, L)]
                    m = x > bs[k]
                    bs[k] = jnp.maximum(bs[k], x)
                    bis[k] = jnp.where(m, cv, bis[k])
                return tuple(bs), tuple(bis), cv + 1

            bests, bidxs, civ = lax.fori_loop(
                0, VECS // K, elem, (bests, bidxs, civ), unroll=4)

        best = bests[0]
        bidx = bidxs[0] * (K * L) + lane
        for k in range(1, K):
            bv = bests[k]
            bi = bidxs[k] * (K * L) + (k * L) + lane
            better = (bv > best) | ((bv == best) & (bi < bidx))
            best = jnp.where(better, bv, best)
            bidx = jnp.where(better, bi, bidx)
        val_mat[r, :] = best
        idx_mat[r, :] = bidx
        return 0

    lax.fori_loop(0, ROWS_PER_W, row_body, 0)

    best = plsc.load_gather(val_mat, [lane, jnp.zeros((L,), jnp.int32)])
    bidx = plsc.load_gather(idx_mat, [lane, jnp.zeros((L,), jnp.int32)])
    for j in range(1, L):
        col = jnp.full((L,), j, jnp.int32)
        bv = plsc.load_gather(val_mat, [lane, col])
        bi = plsc.load_gather(idx_mat, [lane, col])
        better = (bv > best) | ((bv == best) & (bi < bidx))
        best = jnp.where(better, bv, best)
        bidx = jnp.where(better, bi, bidx)
    oid[...] = bidx
    oval[...] = best
    pltpu.sync_copy(oid, id_hbm.at[pl.ds(row0, ROWS_PER_W)])
    pltpu.sync_copy(oval, val_hbm.at[pl.ds(row0, ROWS_PER_W)])


@jax.jit
def kernel(logits):
    x = logits.reshape(R, NCH, CH)
    mesh = plsc.VectorSubcoreMesh(
        core_axis_name="c", subcore_axis_name="s", num_cores=NC, num_subcores=NS)
    run = pl.kernel(
        _sc_body,
        out_type=(
            jax.ShapeDtypeStruct((R,), jnp.int32),
            jax.ShapeDtypeStruct((R,), jnp.float32),
        ),
        mesh=mesh,
        compiler_params=pltpu.CompilerParams(needs_layout_passes=False),
        scratch_types=(
            pltpu.VMEM((CH,), jnp.float32),
            pltpu.VMEM((CH,), jnp.float32),
            pltpu.VMEM((ROWS_PER_W, L), jnp.float32),
            pltpu.VMEM((ROWS_PER_W, L), jnp.int32),
            pltpu.VMEM((ROWS_PER_W,), jnp.int32),
            pltpu.VMEM((ROWS_PER_W,), jnp.float32),
            pltpu.SemaphoreType.DMA,
            pltpu.SemaphoreType.DMA,
        ),
    )
    token_id, token_logit = run(x)
    return token_id.reshape(B, S), token_logit.reshape(B, S)


# speedup vs baseline: 1.1867x; 1.1857x over previous
import jax
import jax.numpy as jnp
from jax import lax
from jax.experimental import pallas as pl
from jax.experimental.pallas import tpu as pltpu
from jax.experimental.pallas import tpu_sc as plsc

B, S, V = 64, 8, 100000
R = B * S
NC, NS, L = 2, 16, 16
NW = NC * NS
B_PER_W = B // NW
ROWS_PER_W = B_PER_W * S
TILE = 128
COLS_MAIN = (V // TILE) * TILE
TAIL = V - COLS_MAIN
CW = 1 * TILE
NCH = COLS_MAIN // CW
K = 4
ITERS = CW // (K * L)

_NEG_INF = float("-inf")


def _sc_body(x_hbm, tail_hbm, id_hbm, val_hbm,
             buf0, buf1, tail_buf, accv, acci, val_mat, idx_mat, oid, oval,
             sem0, sem1, sem_t):
    wid = lax.axis_index("s") * NC + lax.axis_index("c")
    row0 = wid * ROWS_PER_W
    bufs = (buf0, buf1)
    sems = (sem0, sem1)
    lane = lax.broadcasted_iota(jnp.int32, (L,), 0)

    def start(b, w, slot):
        pltpu.make_async_copy(
            x_hbm.at[b, :, pl.ds(w * CW, CW)], bufs[slot], sems[slot]).start()

    def wait(slot):
        pltpu.make_async_copy(
            x_hbm.at[0, :, pl.ds(0, CW)], bufs[slot], sems[slot]).wait()

    def process_chunk(buf, civ0):
        for s in range(S):
            bs = [accv[s, k, :] for k in range(K)]
            bis = [acci[s, k, :] for k in range(K)]

            def it(i, carry):
                cbs, cbis, civ = carry
                cbs, cbis = list(cbs), list(cbis)
                for k in range(K):
                    x = buf[s, pl.ds((i * K + k) * L, L)]
                    m = x > cbs[k]
                    cbs[k] = jnp.maximum(cbs[k], x)
                    cbis[k] = jnp.where(m, civ, cbis[k])
                return tuple(cbs), tuple(cbis), civ + 1

            res = lax.fori_loop(
                0, ITERS, it, (tuple(bs), tuple(bis), civ0), unroll=11)
            for k in range(K):
                accv[s, k, :] = res[0][k]
                acci[s, k, :] = res[1][k]
        return civ0 + ITERS

    for b_local in range(B_PER_W):
        b = wid * B_PER_W + b_local
        for s in range(S):
            for k in range(K):
                accv[s, k, :] = jnp.full((L,), _NEG_INF, jnp.float32)
                acci[s, k, :] = jnp.zeros((L,), jnp.int32)
        start(b, 0, 0)
        start(b, 1, 1)
        tail_cp = pltpu.make_async_copy(tail_hbm.at[b], tail_buf, sem_t)
        tail_cp.start()

        def pair_body(j, civ0):
            for par in range(2):
                w = 2 * j + par
                wait(par)

                @pl.when(w + 2 < NCH)
                def _prefetch():
                    start(b, w + 2, par)

                civ0 = process_chunk(bufs[par], civ0)
            return civ0

        civ0 = lax.fori_loop(
            0, (NCH - 1) // 2, pair_body, jnp.zeros((L,), jnp.int32))
        wait(0)
        civ0 = process_chunk(bufs[0], civ0)

        tail_cp.wait()
        for s in range(S):
            best = accv[s, 0, :]
            bidx = acci[s, 0, :] * (K * L) + lane
            for k in range(1, K):
                bv = accv[s, k, :]
                bi = acci[s, k, :] * (K * L) + (k * L) + lane
                better = (bv > best) | ((bv == best) & (bi < bidx))
                best = jnp.where(better, bv, best)
                bidx = jnp.where(better, bi, bidx)
            for t in range(TILE // L):
                x = tail_buf[s, pl.ds(t * L, L)]
                ci = lane + (COLS_MAIN + t * L)
                m = x > best
                best = jnp.maximum(best, x)
                bidx = jnp.where(m, ci, bidx)
            r = b_local * S + s
            val_mat[r, :] = best
            idx_mat[r, :] = bidx

    best = plsc.load_gather(val_mat, [lane, jnp.zeros((L,), jnp.int32)])
    bidx = plsc.load_gather(idx_mat, [lane, jnp.zeros((L,), jnp.int32)])
    for j in range(1, L):
        col = jnp.full((L,), j, jnp.int32)
        bv = plsc.load_gather(val_mat, [lane, col])
        bi = plsc.load_gather(idx_mat, [lane, col])
        better = (bv > best) | ((bv == best) & (bi < bidx))
        best = jnp.where(better, bv, best)
        bidx = jnp.where(better, bi, bidx)
    oid[...] = bidx
    oval[...] = best
    pltpu.sync_copy(oid, id_hbm.at[pl.ds(row0, ROWS_PER_W)])
    pltpu.sync_copy(oval, val_hbm.at[pl.ds(row0, ROWS_PER_W)])


@jax.jit
def kernel(logits):
    tail = jnp.pad(
        logits[:, :, COLS_MAIN:], ((0, 0), (0, 0), (0, TILE - TAIL)),
        constant_values=_NEG_INF)
    mesh = plsc.VectorSubcoreMesh(
        core_axis_name="c", subcore_axis_name="s", num_cores=NC, num_subcores=NS)
    run = pl.kernel(
        _sc_body,
        out_type=(
            jax.ShapeDtypeStruct((R,), jnp.int32),
            jax.ShapeDtypeStruct((R,), jnp.float32),
        ),
        mesh=mesh,
        compiler_params=pltpu.CompilerParams(needs_layout_passes=False),
        scratch_types=(
            pltpu.VMEM((S, CW), jnp.float32),
            pltpu.VMEM((S, CW), jnp.float32),
            pltpu.VMEM((S, TILE), jnp.float32),
            pltpu.VMEM((S, K, L), jnp.float32),
            pltpu.VMEM((S, K, L), jnp.int32),
            pltpu.VMEM((ROWS_PER_W, L), jnp.float32),
            pltpu.VMEM((ROWS_PER_W, L), jnp.int32),
            pltpu.VMEM((ROWS_PER_W,), jnp.int32),
            pltpu.VMEM((ROWS_PER_W,), jnp.float32),
            pltpu.SemaphoreType.DMA,
            pltpu.SemaphoreType.DMA,
            pltpu.SemaphoreType.DMA,
        ),
    )
    token_id, token_logit = run(logits, tail)
    return token_id.reshape(B, S), token_logit.reshape(B, S)


# speedup vs baseline: 3.8672x; 3.2589x over previous
import jax
import jax.numpy as jnp
from jax import lax
from jax.experimental import pallas as pl
from jax.experimental.pallas import tpu as pltpu
from jax.experimental.pallas import tpu_sc as plsc

B, S, V = 64, 8, 100000
R = B * S
NC, NS, L = 2, 16, 16
NW = NC * NS
B_PER_W = B // NW
ROWS_PER_W = B_PER_W * S
TILE = 128
COLS_MAIN = (V // TILE) * TILE
TAIL = V - COLS_MAIN
NT = 11
CW = NT * TILE
NCH = COLS_MAIN // CW
KC = TILE // L

_NEG_INF = float("-inf")


def _sc_body(x_hbm, tail_hbm, id_hbm, val_hbm,
             buf0, buf1, tail_buf, accv, acci, val_mat, idx_mat, oid, oval,
             sem0, sem1, sem_t):
    wid = lax.axis_index("s") * NC + lax.axis_index("c")
    row0 = wid * ROWS_PER_W
    bufs = (buf0, buf1)
    sems = (sem0, sem1)
    lane = lax.broadcasted_iota(jnp.int32, (L,), 0)

    def start(b, w, slot):
        for t in range(NT):
            pltpu.make_async_copy(
                x_hbm.at[b, :, pl.ds(w * CW + t * TILE, TILE)],
                bufs[slot].at[t], sems[slot]).start()

    def wait(slot):
        for t in range(NT):
            pltpu.make_async_copy(
                x_hbm.at[0, :, pl.ds(0, TILE)],
                bufs[slot].at[0], sems[slot]).wait()

    def process_chunk(buf, civ0):
        def s_body(s, carry):
            bs = [accv[s, c, :] for c in range(KC)]
            bis = [acci[s, c, :] for c in range(KC)]
            for tt in range(NT):
                civ_t = civ0 + tt
                for c in range(KC):
                    x = buf[tt, s, pl.ds(c * L, L)]
                    m = x > bs[c]
                    bs[c] = jnp.maximum(bs[c], x)
                    bis[c] = jnp.where(m, civ_t, bis[c])
            for c in range(KC):
                accv[s, c, :] = bs[c]
                acci[s, c, :] = bis[c]
            return carry

        lax.fori_loop(0, S, s_body, 0)
        return civ0 + NT

    for b_local in range(B_PER_W):
        b = wid * B_PER_W + b_local
        for s in range(S):
            for c in range(KC):
                accv[s, c, :] = jnp.full((L,), _NEG_INF, jnp.float32)
                acci[s, c, :] = jnp.zeros((L,), jnp.int32)
        start(b, 0, 0)
        start(b, 1, 1)
        tail_cp = pltpu.make_async_copy(tail_hbm.at[b], tail_buf, sem_t)
        tail_cp.start()

        def pair_body(j, civ0):
            for par in range(2):
                w = 2 * j + par
                wait(par)
                civ0 = process_chunk(bufs[par], civ0)

                @pl.when(w + 2 < NCH)
                def _prefetch():
                    start(b, w + 2, par)
            return civ0

        civ0 = lax.fori_loop(
            0, (NCH - 1) // 2, pair_body, jnp.zeros((L,), jnp.int32))
        wait(0)
        civ0 = process_chunk(bufs[0], civ0)

        tail_cp.wait()
        for s in range(S):
            best = accv[s, 0, :]
            bidx = acci[s, 0, :] * TILE + lane
            for c in range(1, KC):
                bv = accv[s, c, :]
                bi = acci[s, c, :] * TILE + (c * L) + lane
                better = (bv > best) | ((bv == best) & (bi < bidx))
                best = jnp.where(better, bv, best)
                bidx = jnp.where(better, bi, bidx)
            for t in range(TILE // L):
                x = tail_buf[s, pl.ds(t * L, L)]
                ci = lane + (COLS_MAIN + t * L)
                m = x > best
                best = jnp.maximum(best, x)
                bidx = jnp.where(m, ci, bidx)
            r = b_local * S + s
            val_mat[r, :] = best
            idx_mat[r, :] = bidx

    best = plsc.load_gather(val_mat, [lane, jnp.zeros((L,), jnp.int32)])
    bidx = plsc.load_gather(idx_mat, [lane, jnp.zeros((L,), jnp.int32)])
    for j in range(1, L):
        col = jnp.full((L,), j, jnp.int32)
        bv = plsc.load_gather(val_mat, [lane, col])
        bi = plsc.load_gather(idx_mat, [lane, col])
        better = (bv > best) | ((bv == best) & (bi < bidx))
        best = jnp.where(better, bv, best)
        bidx = jnp.where(better, bi, bidx)
    oid[...] = bidx
    oval[...] = best
    pltpu.sync_copy(oid, id_hbm.at[pl.ds(row0, ROWS_PER_W)])
    pltpu.sync_copy(oval, val_hbm.at[pl.ds(row0, ROWS_PER_W)])


@jax.jit
def kernel(logits):
    tail = jnp.pad(
        logits[:, :, COLS_MAIN:], ((0, 0), (0, 0), (0, TILE - TAIL)),
        constant_values=_NEG_INF)
    mesh = plsc.VectorSubcoreMesh(
        core_axis_name="c", subcore_axis_name="s", num_cores=NC, num_subcores=NS)
    run = pl.kernel(
        _sc_body,
        out_type=(
            jax.ShapeDtypeStruct((R,), jnp.int32),
            jax.ShapeDtypeStruct((R,), jnp.float32),
        ),
        mesh=mesh,
        compiler_params=pltpu.CompilerParams(needs_layout_passes=False),
        scratch_types=(
            pltpu.VMEM((NT, S, TILE), jnp.float32),
            pltpu.VMEM((NT, S, TILE), jnp.float32),
            pltpu.VMEM((S, TILE), jnp.float32),
            pltpu.VMEM((S, KC, L), jnp.float32),
            pltpu.VMEM((S, KC, L), jnp.int32),
            pltpu.VMEM((ROWS_PER_W, L), jnp.float32),
            pltpu.VMEM((ROWS_PER_W, L), jnp.int32),
            pltpu.VMEM((ROWS_PER_W,), jnp.int32),
            pltpu.VMEM((ROWS_PER_W,), jnp.float32),
            pltpu.SemaphoreType.DMA,
            pltpu.SemaphoreType.DMA,
            pltpu.SemaphoreType.DMA,
        ),
    )
    token_id, token_logit = run(logits, tail)
    return token_id.reshape(B, S), token_logit.reshape(B, S)
